# baseline (speedup 1.0000x reference)
import jax
import jax.numpy as jnp
from jax.experimental import pallas as pl
from jax.experimental.pallas import tpu as pltpu

N, K, T, H = 16, 64, 64, 1024


def _body(ids_ref, cxy_ref, wc_ref, bc_ref, wk_ref, bk_ref, pool1_ref, ckm_ref,
          pool0_ref, pmask_ref,
          score_ref, enc_ref, mask_ref, use_ref):
    n = pl.program_id(0)
    idn = ids_ref[n]
    cxy = cxy_ref[0]
    cp = jax.lax.dot_general(cxy, wc_ref[...], (((1,), (1,)), ((), ())),
                             preferred_element_type=jnp.float32)
    cp = cp + bc_ref[...]
    v = jax.lax.dot_general(cp, wk_ref[...], (((1,), (0,)), ((), ())),
                            preferred_element_type=jnp.float32)
    sb = jnp.sum(cp * bk_ref[...])
    p1 = pool1_ref[0]
    sc = jax.lax.dot_general(v, p1, (((1,), (1,)), ((), ())),
                             preferred_element_type=jnp.float32)
    sc = sc + sb
    m = ckm_ref[0]
    sc = jnp.where(m != 0.0, sc, jnp.asarray(-1e20, jnp.float32))
    score_ref[0] = sc
    enc_ref[...] = pool0_ref[0]
    use_ref[0] = pool1_ref[0, pl.ds(idn, 1), :]
    mask_ref[0] = pmask_ref[0, pl.ds(idn, 1), :]


def kernel(contexts_encoded, knowledge_tracking_pool_encoded_0,
           knowledge_tracking_pool_encoded_1, knowledge_tracking_pool_mask,
           tracking_ck_mask, knowledge_tracking_label, Wc, bc, Wk, bk):
    pool0 = knowledge_tracking_pool_encoded_0
    pool1 = knowledge_tracking_pool_encoded_1
    ids = knowledge_tracking_label.astype(jnp.int32)
    ctx = contexts_encoded[1]
    cxy = ctx.reshape(N, 1, 2 * H)
    bc2 = bc.reshape(1, H)
    bk2 = bk.reshape(1, H)
    ckm = tracking_ck_mask.astype(jnp.float32).reshape(N, 1, K)
    pmask = knowledge_tracking_pool_mask.astype(jnp.float32)

    grid_spec = pltpu.PrefetchScalarGridSpec(
        num_scalar_prefetch=1,
        grid=(N,),
        in_specs=[
            pl.BlockSpec((1, 1, 2 * H), lambda n, ids: (n, 0, 0)),
            pl.BlockSpec((H, 2 * H), lambda n, ids: (0, 0)),
            pl.BlockSpec((1, H), lambda n, ids: (0, 0)),
            pl.BlockSpec((H, H), lambda n, ids: (0, 0)),
            pl.BlockSpec((1, H), lambda n, ids: (0, 0)),
            pl.BlockSpec((1, K, H), lambda n, ids: (n, 0, 0)),
            pl.BlockSpec((1, 1, K), lambda n, ids: (n, 0, 0)),
            pl.BlockSpec((1, 1, T, H), lambda n, ids: (n, ids[n], 0, 0)),
            pl.BlockSpec((1, K, T), lambda n, ids: (n, 0, 0)),
        ],
        out_specs=[
            pl.BlockSpec((1, 1, K), lambda n, ids: (n, 0, 0)),
            pl.BlockSpec((1, T, H), lambda n, ids: (n, 0, 0)),
            pl.BlockSpec((1, 1, T), lambda n, ids: (n, 0, 0)),
            pl.BlockSpec((1, 1, H), lambda n, ids: (n, 0, 0)),
        ],
    )
    score3, enc, mask3, use3 = pl.pallas_call(
        _body,
        grid_spec=grid_spec,
        out_shape=[
            jax.ShapeDtypeStruct((N, 1, K), jnp.float32),
            jax.ShapeDtypeStruct((N, T, H), jnp.float32),
            jax.ShapeDtypeStruct((N, 1, T), jnp.float32),
            jax.ShapeDtypeStruct((N, 1, H), jnp.float32),
        ],
    )(ids, cxy, Wc, bc2, Wk, bk2, pool1, ckm, pool0, pmask)

    score = score3.reshape(N, K)
    mask = mask3.reshape(N, T).astype(bool)
    use = use3.reshape(N, H)
    return (score, enc, mask, use)
